# baseline (speedup 1.0000x reference)
import functools

import jax
import jax.numpy as jnp
from jax import lax
from jax.experimental import pallas as pl
from jax.experimental.pallas import tpu as pltpu
from jax.experimental.pallas import tpu_sc as plsc

B = 8
H = 512
W = 512
N = H * W
R = 1024
C = 6
IGNORE = 255
NLANE = 16
NREP = 8
HIST = C * R
TILES_PER_BATCH = 4
ROWS_PER_TILE = H // TILES_PER_BATCH
CHR = 8
CH = CHR * W
HCH = CH // 2
NCHUNK = ROWS_PER_TILE // CHR
WCACHE = ROWS_PER_TILE * W // 2

_mesh = plsc.VectorSubcoreMesh(core_axis_name="c", subcore_axis_name="s")


@functools.partial(
    pl.kernel,
    out_type=jax.ShapeDtypeStruct((B, H, W), jnp.int32),
    mesh=_mesh,
    compiler_params=pltpu.CompilerParams(
        needs_layout_passes=False,
        disable_bounds_checks=True,
        disable_semaphore_checks=True,
    ),
    scratch_types=[
        pltpu.VMEM((NREP * HIST,), jnp.int32),
        pltpu.VMEM((WCACHE,), jnp.int32),
        pltpu.VMEM((2, CHR, W), jnp.int32),
        pltpu.VMEM((2, CHR, W), jnp.int32),
        pltpu.VMEM((2, CHR, W), jnp.int32),
        pltpu.VMEM((R,), jnp.int32),
        pltpu.VMEM_SHARED((16, HIST), jnp.int32),
        pltpu.SemaphoreType.DMA,
        pltpu.SemaphoreType.DMA,
        pltpu.SemaphoreType.DMA,
        pltpu.SemaphoreType.DMA,
        pltpu.SemaphoreType.DMA,
    ],
)
def _homogenize(labels_hbm, regions_hbm, out_hbm,
                hist, wcache, lbuf, rbuf, obuf, maj, shared,
                sem_in0, sem_in1, sem_out0, sem_out1, sem_x):
    cid = lax.axis_index("c")
    sid = lax.axis_index("s")
    batch = cid * 4 + sid // TILES_PER_BATCH
    quarter = sid % TILES_PER_BATCH
    base_row = quarter * ROWS_PER_TILE

    sem_in = (sem_in0, sem_in1)
    sem_out = (sem_out0, sem_out1)
    lanes = lax.iota(jnp.int32, NLANE)
    zeros = jnp.zeros((NLANE,), jnp.int32)
    ones = jnp.full((NLANE,), 1, jnp.int32)
    ign = jnp.full((NLANE,), IGNORE, jnp.int32)
    rep_off = (lanes & (NREP - 1)) * HIST
    mask_lo = lanes < NREP
    mask_hi = lanes >= NREP

    def start_in(ch):
        r0 = base_row + ch * CHR
        slot = ch % 2
        return (
            pltpu.async_copy(labels_hbm.at[batch, pl.ds(r0, CHR)],
                             lbuf.at[slot], sem_in[slot]),
            pltpu.async_copy(regions_hbm.at[batch, pl.ds(r0, CHR)],
                             rbuf.at[slot], sem_in[slot]),
        )

    pend = start_in(0)

    @plsc.parallel_loop(0, NREP * HIST, NLANE, unroll=8)
    def _(o):
        hist[pl.ds(o, NLANE)] = zeros

    for ch in range(NCHUNK):
        slot = ch % 2
        nxt = start_in(ch + 1) if ch + 1 < NCHUNK else ()
        for cp in pend:
            cp.wait()
        pend = nxt

        @plsc.parallel_loop(0, HCH, NLANE, unroll=8)
        def _(o):
            row = o >> 9
            col = o & (W - 1)
            l_a = lbuf[slot, row, pl.ds(col, NLANE)]
            r_a = rbuf[slot, row, pl.ds(col, NLANE)]
            l_b = lbuf[slot, row + CHR // 2, pl.ds(col, NLANE)]
            r_b = rbuf[slot, row + CHR // 2, pl.ds(col, NLANE)]
            w_a = (l_a << 10) + r_a
            w_b = (l_b << 10) + r_b
            plsc.addupdate_scatter(hist, [w_a + rep_off], ones, mask=mask_lo)
            plsc.addupdate_scatter(hist, [w_a + rep_off], ones, mask=mask_hi)
            plsc.addupdate_scatter(hist, [w_b + rep_off], ones, mask=mask_lo)
            plsc.addupdate_scatter(hist, [w_b + rep_off], ones, mask=mask_hi)
            wcache[pl.ds(ch * HCH + o, NLANE)] = w_a + (w_b << 13)

    @plsc.parallel_loop(0, HIST, NLANE, unroll=4)
    def _(o):
        acc = hist[pl.ds(o, NLANE)]
        for rep in range(1, NREP):
            acc = acc + hist[pl.ds(rep * HIST + o, NLANE)]
        hist[pl.ds(o, NLANE)] = acc

    pltpu.sync_copy(hist.at[pl.ds(0, HIST)], shared.at[sid])
    plsc.subcore_barrier()

    gbase = (sid // TILES_PER_BATCH) * TILES_PER_BATCH
    others = [gbase + jnp.where(quarter == p, 0, p)
              for p in range(TILES_PER_BATCH)]
    for cp in [
        pltpu.async_copy(shared.at[others[p]],
                         hist.at[pl.ds(p * HIST, HIST)], sem_x)
        for p in range(1, TILES_PER_BATCH)
    ]:
        cp.wait()

    @plsc.parallel_loop(0, R, NLANE, unroll=2)
    def _(o):
        hs = []
        for c in range(C):
            acc = hist[pl.ds(c * R + o, NLANE)]
            for p in range(1, TILES_PER_BATCH):
                acc = acc + hist[pl.ds(p * HIST + c * R + o, NLANE)]
            hs.append(acc)
        total = hs[0]
        maxv = hs[0]
        for c in range(1, C):
            total = total + hs[c]
            maxv = jnp.maximum(maxv, hs[c])
        amax = jnp.full((NLANE,), C - 1, jnp.int32)
        for c in range(C - 2, -1, -1):
            amax = jnp.where(hs[c] == maxv, jnp.full((NLANE,), c, jnp.int32),
                             amax)
        totf = total.astype(jnp.float32) + jnp.float32(1e-5)
        pct = maxv.astype(jnp.float32) / totf
        maj[pl.ds(o, NLANE)] = jnp.where(pct < jnp.float32(0.9), ign, amax)

    v0 = maj[pl.ds(0, NLANE)]
    maj[pl.ds(0, NLANE)] = jnp.where(lanes == 0, ign, v0)

    pend_out = ((), ())
    for ch in range(NCHUNK):
        slot = ch % 2
        for cp in pend_out[slot]:
            cp.wait()

        @plsc.parallel_loop(0, HCH, NLANE, unroll=8)
        def _(o):
            row = o >> 9
            col = o & (W - 1)
            w = wcache[pl.ds(ch * HCH + o, NLANE)]
            w_a = w & 0x1FFF
            w_b = w >> 13
            r_a = w_a & (R - 1)
            l_a = w_a >> 10
            r_b = w_b & (R - 1)
            l_b = w_b >> 10
            g_a = plsc.load_gather(maj, [r_a])
            g_b = plsc.load_gather(maj, [r_b])
            obuf[slot, row, pl.ds(col, NLANE)] = jnp.where(g_a == ign, l_a, g_a)
            obuf[slot, row + CHR // 2, pl.ds(col, NLANE)] = jnp.where(
                g_b == ign, l_b, g_b)

        r0 = base_row + ch * CHR
        out_cp = pltpu.async_copy(obuf.at[slot],
                                  out_hbm.at[batch, pl.ds(r0, CHR)],
                                  sem_out[slot])
        pend_out = tuple(
            (out_cp,) if s == slot else pend_out[s] for s in range(2))

    for s in range(2):
        for cp in pend_out[s]:
            cp.wait()


def kernel(pseudo_labels, regions):
    return _homogenize(pseudo_labels, regions)


# speedup vs baseline: 1.0375x; 1.0375x over previous
import functools

import jax
import jax.numpy as jnp
from jax import lax
from jax.experimental import pallas as pl
from jax.experimental.pallas import tpu as pltpu
from jax.experimental.pallas import tpu_sc as plsc

B = 8
H = 512
W = 512
N = H * W
R = 1024
C = 6
IGNORE = 255
NLANE = 16
NREP = 8
HIST = C * R
TILES_PER_BATCH = 4
ROWS_PER_TILE = H // TILES_PER_BATCH
CHR = 8
CH = CHR * W
HCH = CH // 2
NCHUNK = ROWS_PER_TILE // CHR
WCACHE = ROWS_PER_TILE * W // 2

_mesh = plsc.VectorSubcoreMesh(core_axis_name="c", subcore_axis_name="s")


@functools.partial(
    pl.kernel,
    out_type=jax.ShapeDtypeStruct((B, H, W), jnp.int32),
    mesh=_mesh,
    compiler_params=pltpu.CompilerParams(
        needs_layout_passes=False,
        disable_bounds_checks=True,
        disable_semaphore_checks=True,
    ),
    scratch_types=[
        pltpu.VMEM((NREP * HIST,), jnp.int32),
        pltpu.VMEM((WCACHE,), jnp.int32),
        pltpu.VMEM((2, CHR, W), jnp.int32),
        pltpu.VMEM((2, CHR, W), jnp.int32),
        pltpu.VMEM((2, CHR, W), jnp.int32),
        pltpu.VMEM((R,), jnp.int32),
        pltpu.VMEM_SHARED((16, HIST), jnp.int32),
        pltpu.SemaphoreType.DMA,
        pltpu.SemaphoreType.DMA,
        pltpu.SemaphoreType.DMA,
        pltpu.SemaphoreType.DMA,
        pltpu.SemaphoreType.DMA,
    ],
)
def _homogenize(labels_hbm, regions_hbm, out_hbm,
                hist, wcache, lbuf, rbuf, obuf, maj, shared,
                sem_in0, sem_in1, sem_out0, sem_out1, sem_x):
    cid = lax.axis_index("c")
    sid = lax.axis_index("s")
    batch = cid * 4 + sid // TILES_PER_BATCH
    quarter = sid % TILES_PER_BATCH
    base_row = quarter * ROWS_PER_TILE

    sem_in = (sem_in0, sem_in1)
    sem_out = (sem_out0, sem_out1)
    lanes = lax.iota(jnp.int32, NLANE)
    zeros = jnp.zeros((NLANE,), jnp.int32)
    ones = jnp.full((NLANE,), 1, jnp.int32)
    ign = jnp.full((NLANE,), IGNORE, jnp.int32)
    rep_off = (lanes & (NREP - 1)) * HIST
    mask_lo = lanes < NREP
    mask_hi = lanes >= NREP

    def start_in(ch):
        r0 = base_row + ch * CHR
        slot = ch % 2
        return (
            pltpu.async_copy(labels_hbm.at[batch, pl.ds(r0, CHR)],
                             lbuf.at[slot], sem_in[slot]),
            pltpu.async_copy(regions_hbm.at[batch, pl.ds(r0, CHR)],
                             rbuf.at[slot], sem_in[slot]),
        )

    pend = start_in(0)

    @plsc.parallel_loop(0, NREP * HIST, NLANE, unroll=8)
    def _(o):
        hist[pl.ds(o, NLANE)] = zeros

    for ch in range(NCHUNK):
        slot = ch % 2
        nxt = start_in(ch + 1) if ch + 1 < NCHUNK else ()
        for cp in pend:
            cp.wait()
        pend = nxt

        @plsc.parallel_loop(0, HCH, NLANE, unroll=4)
        def _(o):
            row = o >> 9
            col = o & (W - 1)
            l_a = lbuf[slot, row, pl.ds(col, NLANE)]
            r_a = rbuf[slot, row, pl.ds(col, NLANE)]
            l_b = lbuf[slot, row + CHR // 2, pl.ds(col, NLANE)]
            r_b = rbuf[slot, row + CHR // 2, pl.ds(col, NLANE)]
            w_a = (l_a << 10) + r_a
            w_b = (l_b << 10) + r_b
            plsc.addupdate_scatter(hist, [w_a + rep_off], ones, mask=mask_lo)
            plsc.addupdate_scatter(hist, [w_a + rep_off], ones, mask=mask_hi)
            plsc.addupdate_scatter(hist, [w_b + rep_off], ones, mask=mask_lo)
            plsc.addupdate_scatter(hist, [w_b + rep_off], ones, mask=mask_hi)
            wcache[pl.ds(ch * HCH + o, NLANE)] = w_a + (w_b << 13)

    @plsc.parallel_loop(0, HIST, NLANE, unroll=4)
    def _(o):
        acc = hist[pl.ds(o, NLANE)]
        for rep in range(1, NREP):
            acc = acc + hist[pl.ds(rep * HIST + o, NLANE)]
        hist[pl.ds(o, NLANE)] = acc

    pltpu.sync_copy(hist.at[pl.ds(0, HIST)], shared.at[sid])
    plsc.subcore_barrier()

    gbase = (sid // TILES_PER_BATCH) * TILES_PER_BATCH
    others = [gbase + jnp.where(quarter == p, 0, p)
              for p in range(TILES_PER_BATCH)]
    for cp in [
        pltpu.async_copy(shared.at[others[p]],
                         hist.at[pl.ds(p * HIST, HIST)], sem_x)
        for p in range(1, TILES_PER_BATCH)
    ]:
        cp.wait()

    @plsc.parallel_loop(0, R, NLANE, unroll=2)
    def _(o):
        hs = []
        for c in range(C):
            acc = hist[pl.ds(c * R + o, NLANE)]
            for p in range(1, TILES_PER_BATCH):
                acc = acc + hist[pl.ds(p * HIST + c * R + o, NLANE)]
            hs.append(acc)
        total = hs[0]
        maxv = hs[0]
        for c in range(1, C):
            total = total + hs[c]
            maxv = jnp.maximum(maxv, hs[c])
        amax = jnp.full((NLANE,), C - 1, jnp.int32)
        for c in range(C - 2, -1, -1):
            amax = jnp.where(hs[c] == maxv, jnp.full((NLANE,), c, jnp.int32),
                             amax)
        totf = total.astype(jnp.float32) + jnp.float32(1e-5)
        pct = maxv.astype(jnp.float32) / totf
        maj[pl.ds(o, NLANE)] = jnp.where(pct < jnp.float32(0.9), ign, amax)

    v0 = maj[pl.ds(0, NLANE)]
    maj[pl.ds(0, NLANE)] = jnp.where(lanes == 0, ign, v0)

    pend_out = ((), ())
    for ch in range(NCHUNK):
        slot = ch % 2
        for cp in pend_out[slot]:
            cp.wait()

        @plsc.parallel_loop(0, HCH, NLANE, unroll=4)
        def _(o):
            row = o >> 9
            col = o & (W - 1)
            w = wcache[pl.ds(ch * HCH + o, NLANE)]
            w_a = w & 0x1FFF
            w_b = w >> 13
            r_a = w_a & (R - 1)
            l_a = w_a >> 10
            r_b = w_b & (R - 1)
            l_b = w_b >> 10
            g_a = plsc.load_gather(maj, [r_a])
            g_b = plsc.load_gather(maj, [r_b])
            obuf[slot, row, pl.ds(col, NLANE)] = jnp.where(g_a == ign, l_a, g_a)
            obuf[slot, row + CHR // 2, pl.ds(col, NLANE)] = jnp.where(
                g_b == ign, l_b, g_b)

        r0 = base_row + ch * CHR
        out_cp = pltpu.async_copy(obuf.at[slot],
                                  out_hbm.at[batch, pl.ds(r0, CHR)],
                                  sem_out[slot])
        pend_out = tuple(
            (out_cp,) if s == slot else pend_out[s] for s in range(2))

    for s in range(2):
        for cp in pend_out[s]:
            cp.wait()


def kernel(pseudo_labels, regions):
    return _homogenize(pseudo_labels, regions)


# speedup vs baseline: 1.0376x; 1.0002x over previous
import functools

import jax
import jax.numpy as jnp
from jax import lax
from jax.experimental import pallas as pl
from jax.experimental.pallas import tpu as pltpu
from jax.experimental.pallas import tpu_sc as plsc

B = 8
H = 512
W = 512
N = H * W
R = 1024
C = 6
IGNORE = 255
NLANE = 16
NREP = 8
HIST = C * R
TILES_PER_BATCH = 4
ROWS_PER_TILE = H // TILES_PER_BATCH
CHR = 8
CH = CHR * W
HCH = CH // 2
NCHUNK = ROWS_PER_TILE // CHR
WCACHE = ROWS_PER_TILE * W // 2

_mesh = plsc.VectorSubcoreMesh(core_axis_name="c", subcore_axis_name="s")


@functools.partial(
    pl.kernel,
    out_type=jax.ShapeDtypeStruct((B, H, W), jnp.int32),
    mesh=_mesh,
    compiler_params=pltpu.CompilerParams(
        needs_layout_passes=False,
        disable_bounds_checks=True,
        disable_semaphore_checks=True,
    ),
    scratch_types=[
        pltpu.VMEM((NREP * HIST,), jnp.int32),
        pltpu.VMEM((WCACHE,), jnp.int32),
        pltpu.VMEM((2, CHR, W), jnp.int32),
        pltpu.VMEM((2, CHR, W), jnp.int32),
        pltpu.VMEM((2, CHR, W), jnp.int32),
        pltpu.VMEM((R,), jnp.int32),
        pltpu.VMEM_SHARED((16, HIST), jnp.int32),
        pltpu.SemaphoreType.DMA,
        pltpu.SemaphoreType.DMA,
        pltpu.SemaphoreType.DMA,
        pltpu.SemaphoreType.DMA,
        pltpu.SemaphoreType.DMA,
    ],
)
def _homogenize(labels_hbm, regions_hbm, out_hbm,
                hist, wcache, lbuf, rbuf, obuf, maj, shared,
                sem_in0, sem_in1, sem_out0, sem_out1, sem_x):
    cid = lax.axis_index("c")
    sid = lax.axis_index("s")
    batch = cid * 4 + sid // TILES_PER_BATCH
    quarter = sid % TILES_PER_BATCH
    base_row = quarter * ROWS_PER_TILE

    sem_in = (sem_in0, sem_in1)
    sem_out = (sem_out0, sem_out1)
    lanes = lax.iota(jnp.int32, NLANE)
    zeros = jnp.zeros((NLANE,), jnp.int32)
    ones = jnp.full((NLANE,), 1, jnp.int32)
    ign = jnp.full((NLANE,), IGNORE, jnp.int32)
    rep_off = (lanes & (NREP - 1)) * HIST
    mask_lo = lanes < NREP
    mask_hi = lanes >= NREP

    def start_in(ch):
        r0 = base_row + ch * CHR
        slot = ch % 2
        return (
            pltpu.async_copy(labels_hbm.at[batch, pl.ds(r0, CHR)],
                             lbuf.at[slot], sem_in[slot]),
            pltpu.async_copy(regions_hbm.at[batch, pl.ds(r0, CHR)],
                             rbuf.at[slot], sem_in[slot]),
        )

    pend = start_in(0)

    with jax.named_scope("ph_zero"):
        @plsc.parallel_loop(0, NREP * HIST, NLANE, unroll=8)
        def _(o):
            hist[pl.ds(o, NLANE)] = zeros

    scopeA = jax.named_scope("ph_A"); scopeA.__enter__()
    for ch in range(NCHUNK):
        slot = ch % 2
        nxt = start_in(ch + 1) if ch + 1 < NCHUNK else ()
        for cp in pend:
            cp.wait()
        pend = nxt

        @plsc.parallel_loop(0, HCH, NLANE, unroll=4)
        def _(o):
            row = o >> 9
            col = o & (W - 1)
            l_a = lbuf[slot, row, pl.ds(col, NLANE)]
            r_a = rbuf[slot, row, pl.ds(col, NLANE)]
            l_b = lbuf[slot, row + CHR // 2, pl.ds(col, NLANE)]
            r_b = rbuf[slot, row + CHR // 2, pl.ds(col, NLANE)]
            w_a = (l_a << 10) + r_a
            w_b = (l_b << 10) + r_b
            plsc.addupdate_scatter(hist, [w_a + rep_off], ones, mask=mask_lo)
            plsc.addupdate_scatter(hist, [w_a + rep_off], ones, mask=mask_hi)
            plsc.addupdate_scatter(hist, [w_b + rep_off], ones, mask=mask_lo)
            plsc.addupdate_scatter(hist, [w_b + rep_off], ones, mask=mask_hi)
            wcache[pl.ds(ch * HCH + o, NLANE)] = w_a + (w_b << 13)

    scopeA.__exit__(None, None, None)
    scopeR = jax.named_scope("ph_red"); scopeR.__enter__()
    @plsc.parallel_loop(0, HIST, NLANE, unroll=4)
    def _(o):
        acc = hist[pl.ds(o, NLANE)]
        for rep in range(1, NREP):
            acc = acc + hist[pl.ds(rep * HIST + o, NLANE)]
        hist[pl.ds(o, NLANE)] = acc

    scopeR.__exit__(None, None, None)
    scopeX = jax.named_scope("ph_xchg"); scopeX.__enter__()
    pltpu.sync_copy(hist.at[pl.ds(0, HIST)], shared.at[sid])
    plsc.subcore_barrier()

    gbase = (sid // TILES_PER_BATCH) * TILES_PER_BATCH
    others = [gbase + jnp.where(quarter == p, 0, p)
              for p in range(TILES_PER_BATCH)]
    for cp in [
        pltpu.async_copy(shared.at[others[p]],
                         hist.at[pl.ds(p * HIST, HIST)], sem_x)
        for p in range(1, TILES_PER_BATCH)
    ]:
        cp.wait()

    scopeX.__exit__(None, None, None)
    scopeB = jax.named_scope("ph_B"); scopeB.__enter__()
    @plsc.parallel_loop(0, R, NLANE, unroll=2)
    def _(o):
        hs = []
        for c in range(C):
            acc = hist[pl.ds(c * R + o, NLANE)]
            for p in range(1, TILES_PER_BATCH):
                acc = acc + hist[pl.ds(p * HIST + c * R + o, NLANE)]
            hs.append(acc)
        total = hs[0]
        maxv = hs[0]
        for c in range(1, C):
            total = total + hs[c]
            maxv = jnp.maximum(maxv, hs[c])
        amax = jnp.full((NLANE,), C - 1, jnp.int32)
        for c in range(C - 2, -1, -1):
            amax = jnp.where(hs[c] == maxv, jnp.full((NLANE,), c, jnp.int32),
                             amax)
        totf = total.astype(jnp.float32) + jnp.float32(1e-5)
        pct = maxv.astype(jnp.float32) / totf
        maj[pl.ds(o, NLANE)] = jnp.where(pct < jnp.float32(0.9), ign, amax)

    v0 = maj[pl.ds(0, NLANE)]
    maj[pl.ds(0, NLANE)] = jnp.where(lanes == 0, ign, v0)

    scopeB.__exit__(None, None, None)
    scopeC = jax.named_scope("ph_C"); scopeC.__enter__()
    pend_out = ((), ())
    for ch in range(NCHUNK):
        slot = ch % 2
        for cp in pend_out[slot]:
            cp.wait()

        @plsc.parallel_loop(0, HCH, NLANE, unroll=4)
        def _(o):
            row = o >> 9
            col = o & (W - 1)
            w = wcache[pl.ds(ch * HCH + o, NLANE)]
            w_a = w & 0x1FFF
            w_b = w >> 13
            r_a = w_a & (R - 1)
            l_a = w_a >> 10
            r_b = w_b & (R - 1)
            l_b = w_b >> 10
            g_a = plsc.load_gather(maj, [r_a])
            g_b = plsc.load_gather(maj, [r_b])
            obuf[slot, row, pl.ds(col, NLANE)] = jnp.where(g_a == ign, l_a, g_a)
            obuf[slot, row + CHR // 2, pl.ds(col, NLANE)] = jnp.where(
                g_b == ign, l_b, g_b)

        r0 = base_row + ch * CHR
        out_cp = pltpu.async_copy(obuf.at[slot],
                                  out_hbm.at[batch, pl.ds(r0, CHR)],
                                  sem_out[slot])
        pend_out = tuple(
            (out_cp,) if s == slot else pend_out[s] for s in range(2))

    for s in range(2):
        for cp in pend_out[s]:
            cp.wait()
    scopeC.__exit__(None, None, None)


SCOPE_END_MARKER = None


def kernel(pseudo_labels, regions):
    return _homogenize(pseudo_labels, regions)


# speedup vs baseline: 1.0457x; 1.0077x over previous
import functools

import jax
import jax.numpy as jnp
from jax import lax
from jax.experimental import pallas as pl
from jax.experimental.pallas import tpu as pltpu
from jax.experimental.pallas import tpu_sc as plsc

B = 8
H = 512
W = 512
N = H * W
R = 1024
C = 6
IGNORE = 255
NLANE = 16
NREP = 8
HIST = C * R
TILES_PER_BATCH = 4
ROWS_PER_TILE = H // TILES_PER_BATCH
CHR = 8
CH = CHR * W
HCH = CH // 2
NCHUNK = ROWS_PER_TILE // CHR
WCACHE = ROWS_PER_TILE * W // 2

_mesh = plsc.VectorSubcoreMesh(core_axis_name="c", subcore_axis_name="s")


@functools.partial(
    pl.kernel,
    out_type=jax.ShapeDtypeStruct((B, H, W), jnp.int32),
    mesh=_mesh,
    compiler_params=pltpu.CompilerParams(
        needs_layout_passes=False,
        disable_bounds_checks=True,
        disable_semaphore_checks=True,
    ),
    scratch_types=[
        pltpu.VMEM((NREP * HIST,), jnp.int32),
        pltpu.VMEM((WCACHE,), jnp.int32),
        pltpu.VMEM((2, CHR, W), jnp.int32),
        pltpu.VMEM((2, CHR, W), jnp.int32),
        pltpu.VMEM((2, CHR, W), jnp.int32),
        pltpu.VMEM((R,), jnp.int32),
        pltpu.VMEM_SHARED((16, HIST), jnp.int32),
        pltpu.SemaphoreType.DMA,
        pltpu.SemaphoreType.DMA,
        pltpu.SemaphoreType.DMA,
        pltpu.SemaphoreType.DMA,
        pltpu.SemaphoreType.DMA,
    ],
)
def _homogenize(labels_hbm, regions_hbm, out_hbm,
                hist, wcache, lbuf, rbuf, obuf, maj, shared,
                sem_in0, sem_in1, sem_out0, sem_out1, sem_x):
    cid = lax.axis_index("c")
    sid = lax.axis_index("s")
    batch = cid * 4 + sid // TILES_PER_BATCH
    quarter = sid % TILES_PER_BATCH
    base_row = quarter * ROWS_PER_TILE

    sem_in = (sem_in0, sem_in1)
    sem_out = (sem_out0, sem_out1)
    lanes = lax.iota(jnp.int32, NLANE)
    zeros = jnp.zeros((NLANE,), jnp.int32)
    ones = jnp.full((NLANE,), 1, jnp.int32)
    ign = jnp.full((NLANE,), IGNORE, jnp.int32)
    rep_off = (lanes & (NREP - 1)) * HIST
    mask_lo = lanes < NREP
    mask_hi = lanes >= NREP

    def start_in(ch):
        r0 = base_row + ch * CHR
        slot = ch % 2
        return (
            pltpu.async_copy(labels_hbm.at[batch, pl.ds(r0, CHR)],
                             lbuf.at[slot], sem_in[slot]),
            pltpu.async_copy(regions_hbm.at[batch, pl.ds(r0, CHR)],
                             rbuf.at[slot], sem_in[slot]),
        )

    pend = start_in(0)

    with jax.named_scope("ph_zero"):
        @plsc.parallel_loop(0, NREP * HIST, NLANE, unroll=8)
        def _(o):
            hist[pl.ds(o, NLANE)] = zeros

    scopeA = jax.named_scope("ph_A"); scopeA.__enter__()
    for ch in range(NCHUNK):
        slot = ch % 2
        nxt = start_in(ch + 1) if ch + 1 < NCHUNK else ()
        for cp in pend:
            cp.wait()
        pend = nxt

        @plsc.parallel_loop(0, CH, 2 * NLANE, unroll=4)
        def _(o):
            row = o >> 9
            col = o & (W - 1)
            l_a = lbuf[slot, row, pl.ds(col, NLANE)]
            r_a = rbuf[slot, row, pl.ds(col, NLANE)]
            l_b = lbuf[slot, row, pl.ds(col + NLANE, NLANE)]
            r_b = rbuf[slot, row, pl.ds(col + NLANE, NLANE)]
            w_a = (l_a << 10) + r_a
            w_b = (l_b << 10) + r_b
            plsc.addupdate_scatter(hist, [w_a + rep_off], ones, mask=mask_lo)
            plsc.addupdate_scatter(hist, [w_a + rep_off], ones, mask=mask_hi)
            plsc.addupdate_scatter(hist, [w_b + rep_off], ones, mask=mask_lo)
            plsc.addupdate_scatter(hist, [w_b + rep_off], ones, mask=mask_hi)
            wcache[pl.ds(ch * HCH + (o >> 1), NLANE)] = w_a + (w_b << 13)

    scopeA.__exit__(None, None, None)
    scopeR = jax.named_scope("ph_red"); scopeR.__enter__()
    @plsc.parallel_loop(0, HIST, NLANE, unroll=4)
    def _(o):
        acc = hist[pl.ds(o, NLANE)]
        for rep in range(1, NREP):
            acc = acc + hist[pl.ds(rep * HIST + o, NLANE)]
        hist[pl.ds(o, NLANE)] = acc

    scopeR.__exit__(None, None, None)
    scopeX = jax.named_scope("ph_xchg"); scopeX.__enter__()
    pltpu.sync_copy(hist.at[pl.ds(0, HIST)], shared.at[sid])
    plsc.subcore_barrier()

    gbase = (sid // TILES_PER_BATCH) * TILES_PER_BATCH
    others = [gbase + jnp.where(quarter == p, 0, p)
              for p in range(TILES_PER_BATCH)]
    for cp in [
        pltpu.async_copy(shared.at[others[p]],
                         hist.at[pl.ds(p * HIST, HIST)], sem_x)
        for p in range(1, TILES_PER_BATCH)
    ]:
        cp.wait()

    scopeX.__exit__(None, None, None)
    scopeB = jax.named_scope("ph_B"); scopeB.__enter__()
    @plsc.parallel_loop(0, R, NLANE, unroll=2)
    def _(o):
        hs = []
        for c in range(C):
            acc = hist[pl.ds(c * R + o, NLANE)]
            for p in range(1, TILES_PER_BATCH):
                acc = acc + hist[pl.ds(p * HIST + c * R + o, NLANE)]
            hs.append(acc)
        total = hs[0]
        maxv = hs[0]
        for c in range(1, C):
            total = total + hs[c]
            maxv = jnp.maximum(maxv, hs[c])
        amax = jnp.full((NLANE,), C - 1, jnp.int32)
        for c in range(C - 2, -1, -1):
            amax = jnp.where(hs[c] == maxv, jnp.full((NLANE,), c, jnp.int32),
                             amax)
        totf = total.astype(jnp.float32) + jnp.float32(1e-5)
        pct = maxv.astype(jnp.float32) / totf
        maj[pl.ds(o, NLANE)] = jnp.where(pct < jnp.float32(0.9), ign, amax)

    v0 = maj[pl.ds(0, NLANE)]
    maj[pl.ds(0, NLANE)] = jnp.where(lanes == 0, ign, v0)

    scopeB.__exit__(None, None, None)
    scopeC = jax.named_scope("ph_C"); scopeC.__enter__()
    pend_out = ((), ())
    for ch in range(NCHUNK):
        slot = ch % 2
        for cp in pend_out[slot]:
            cp.wait()

        @plsc.parallel_loop(0, HCH, NLANE, unroll=4)
        def _(k):
            row = k >> 8
            col = (k & (W // 2 - 1)) << 1
            w = wcache[pl.ds(ch * HCH + k, NLANE)]
            w_a = w & 0x1FFF
            w_b = w >> 13
            r_a = w_a & (R - 1)
            l_a = w_a >> 10
            r_b = w_b & (R - 1)
            l_b = w_b >> 10
            g_a = plsc.load_gather(maj, [r_a])
            g_b = plsc.load_gather(maj, [r_b])
            obuf[slot, row, pl.ds(col, NLANE)] = jnp.where(g_a == ign, l_a, g_a)
            obuf[slot, row, pl.ds(col + NLANE, NLANE)] = jnp.where(
                g_b == ign, l_b, g_b)

        r0 = base_row + ch * CHR
        out_cp = pltpu.async_copy(obuf.at[slot],
                                  out_hbm.at[batch, pl.ds(r0, CHR)],
                                  sem_out[slot])
        pend_out = tuple(
            (out_cp,) if s == slot else pend_out[s] for s in range(2))

    for s in range(2):
        for cp in pend_out[s]:
            cp.wait()
    scopeC.__exit__(None, None, None)


SCOPE_END_MARKER = None


def kernel(pseudo_labels, regions):
    return _homogenize(pseudo_labels, regions)


# speedup vs baseline: 1.1113x; 1.0628x over previous
import functools

import jax
import jax.numpy as jnp
from jax import lax
from jax.experimental import pallas as pl
from jax.experimental.pallas import tpu as pltpu
from jax.experimental.pallas import tpu_sc as plsc

B = 8
H = 512
W = 512
N = H * W
R = 1024
C = 6
IGNORE = 255
NLANE = 16
NREP = 8
HIST = C * R
TILES_PER_BATCH = 4
ROWS_PER_TILE = H // TILES_PER_BATCH
CHR = 8
CH = CHR * W
HCH = CH // 2
NCHUNK = ROWS_PER_TILE // CHR
WCACHE = ROWS_PER_TILE * W // 2

_mesh = plsc.VectorSubcoreMesh(core_axis_name="c", subcore_axis_name="s")


@functools.partial(
    pl.kernel,
    out_type=jax.ShapeDtypeStruct((B, H, W), jnp.int32),
    mesh=_mesh,
    compiler_params=pltpu.CompilerParams(
        needs_layout_passes=False,
        disable_bounds_checks=True,
        disable_semaphore_checks=True,
    ),
    scratch_types=[
        pltpu.VMEM((NREP * HIST,), jnp.int32),
        pltpu.VMEM((WCACHE,), jnp.int32),
        pltpu.VMEM((4, CHR, W), jnp.int32),
        pltpu.VMEM((4, CHR, W), jnp.int32),
        pltpu.VMEM((2, CHR, W), jnp.int32),
        pltpu.VMEM((R,), jnp.int32),
        pltpu.VMEM_SHARED((16, HIST), jnp.int32),
        pltpu.SemaphoreType.DMA,
        pltpu.SemaphoreType.DMA,
        pltpu.SemaphoreType.DMA,
        pltpu.SemaphoreType.DMA,
        pltpu.SemaphoreType.DMA,
        pltpu.SemaphoreType.DMA,
        pltpu.SemaphoreType.DMA,
    ],
)
def _homogenize(labels_hbm, regions_hbm, out_hbm,
                hist, wcache, lbuf, rbuf, obuf, maj, shared,
                sem_in0, sem_in1, sem_in2, sem_in3,
                sem_out0, sem_out1, sem_x):
    cid = lax.axis_index("c")
    sid = lax.axis_index("s")
    batch = cid * 4 + sid // TILES_PER_BATCH
    quarter = sid % TILES_PER_BATCH
    base_row = quarter * ROWS_PER_TILE

    sem_in = (sem_in0, sem_in1, sem_in2, sem_in3)
    sem_out = (sem_out0, sem_out1)
    lanes = lax.iota(jnp.int32, NLANE)
    zeros = jnp.zeros((NLANE,), jnp.int32)
    ones = jnp.full((NLANE,), 1, jnp.int32)
    ign = jnp.full((NLANE,), IGNORE, jnp.int32)
    rep_off = (lanes & (NREP - 1)) * HIST
    mask_lo = lanes < NREP
    mask_hi = lanes >= NREP

    def start_in(ch):
        r0 = base_row + ch * CHR
        slot = ch % 4
        return (
            pltpu.async_copy(labels_hbm.at[batch, pl.ds(r0, CHR)],
                             lbuf.at[slot], sem_in[slot]),
            pltpu.async_copy(regions_hbm.at[batch, pl.ds(r0, CHR)],
                             rbuf.at[slot], sem_in[slot]),
        )

    pend = [start_in(c) for c in range(3)]

    with jax.named_scope("ph_zero"):
        @plsc.parallel_loop(0, NREP * HIST, NLANE, unroll=8)
        def _(o):
            hist[pl.ds(o, NLANE)] = zeros

    scopeA = jax.named_scope("ph_A"); scopeA.__enter__()
    for ch in range(NCHUNK):
        slot = ch % 4
        if ch + 3 < NCHUNK:
            pend.append(start_in(ch + 3))
        for cp in pend.pop(0):
            cp.wait()

        @plsc.parallel_loop(0, CH, 2 * NLANE, unroll=4)
        def _(o):
            row = o >> 9
            col = o & (W - 1)
            l_a = lbuf[slot, row, pl.ds(col, NLANE)]
            r_a = rbuf[slot, row, pl.ds(col, NLANE)]
            l_b = lbuf[slot, row, pl.ds(col + NLANE, NLANE)]
            r_b = rbuf[slot, row, pl.ds(col + NLANE, NLANE)]
            w_a = (l_a << 10) + r_a
            w_b = (l_b << 10) + r_b
            plsc.addupdate_scatter(hist, [w_a + rep_off], ones, mask=mask_lo)
            plsc.addupdate_scatter(hist, [w_a + rep_off], ones, mask=mask_hi)
            plsc.addupdate_scatter(hist, [w_b + rep_off], ones, mask=mask_lo)
            plsc.addupdate_scatter(hist, [w_b + rep_off], ones, mask=mask_hi)
            wcache[pl.ds(ch * HCH + (o >> 1), NLANE)] = w_a + (w_b << 13)

    scopeA.__exit__(None, None, None)
    scopeR = jax.named_scope("ph_red"); scopeR.__enter__()
    @plsc.parallel_loop(0, HIST, NLANE, unroll=4)
    def _(o):
        acc = hist[pl.ds(o, NLANE)]
        for rep in range(1, NREP):
            acc = acc + hist[pl.ds(rep * HIST + o, NLANE)]
        hist[pl.ds(o, NLANE)] = acc

    scopeR.__exit__(None, None, None)
    scopeX = jax.named_scope("ph_xchg"); scopeX.__enter__()
    pltpu.sync_copy(hist.at[pl.ds(0, HIST)], shared.at[sid])
    plsc.subcore_barrier()

    gbase = (sid // TILES_PER_BATCH) * TILES_PER_BATCH
    others = [gbase + jnp.where(quarter == p, 0, p)
              for p in range(TILES_PER_BATCH)]
    for cp in [
        pltpu.async_copy(shared.at[others[p]],
                         hist.at[pl.ds(p * HIST, HIST)], sem_x)
        for p in range(1, TILES_PER_BATCH)
    ]:
        cp.wait()

    scopeX.__exit__(None, None, None)
    scopeB = jax.named_scope("ph_B"); scopeB.__enter__()
    @plsc.parallel_loop(0, R, NLANE, unroll=2)
    def _(o):
        hs = []
        for c in range(C):
            acc = hist[pl.ds(c * R + o, NLANE)]
            for p in range(1, TILES_PER_BATCH):
                acc = acc + hist[pl.ds(p * HIST + c * R + o, NLANE)]
            hs.append(acc)
        total = hs[0]
        maxv = hs[0]
        for c in range(1, C):
            total = total + hs[c]
            maxv = jnp.maximum(maxv, hs[c])
        amax = jnp.full((NLANE,), C - 1, jnp.int32)
        for c in range(C - 2, -1, -1):
            amax = jnp.where(hs[c] == maxv, jnp.full((NLANE,), c, jnp.int32),
                             amax)
        totf = total.astype(jnp.float32) + jnp.float32(1e-5)
        pct = maxv.astype(jnp.float32) / totf
        maj[pl.ds(o, NLANE)] = jnp.where(pct < jnp.float32(0.9), ign, amax)

    v0 = maj[pl.ds(0, NLANE)]
    maj[pl.ds(0, NLANE)] = jnp.where(lanes == 0, ign, v0)

    scopeB.__exit__(None, None, None)
    scopeC = jax.named_scope("ph_C"); scopeC.__enter__()
    pend_out = ((), ())
    for ch in range(NCHUNK):
        slot = ch % 2
        for cp in pend_out[slot]:
            cp.wait()

        @plsc.parallel_loop(0, HCH, NLANE, unroll=4)
        def _(k):
            row = k >> 8
            col = (k & (W // 2 - 1)) << 1
            w = wcache[pl.ds(ch * HCH + k, NLANE)]
            w_a = w & 0x1FFF
            w_b = w >> 13
            r_a = w_a & (R - 1)
            l_a = w_a >> 10
            r_b = w_b & (R - 1)
            l_b = w_b >> 10
            g_a = plsc.load_gather(maj, [r_a])
            g_b = plsc.load_gather(maj, [r_b])
            obuf[slot, row, pl.ds(col, NLANE)] = jnp.where(g_a == ign, l_a, g_a)
            obuf[slot, row, pl.ds(col + NLANE, NLANE)] = jnp.where(
                g_b == ign, l_b, g_b)

        r0 = base_row + ch * CHR
        out_cp = pltpu.async_copy(obuf.at[slot],
                                  out_hbm.at[batch, pl.ds(r0, CHR)],
                                  sem_out[slot])
        pend_out = tuple(
            (out_cp,) if s == slot else pend_out[s] for s in range(2))

    for s in range(2):
        for cp in pend_out[s]:
            cp.wait()
    scopeC.__exit__(None, None, None)


SCOPE_END_MARKER = None


def kernel(pseudo_labels, regions):
    return _homogenize(pseudo_labels, regions)
